# baseline (speedup 1.0000x reference)
import functools

import jax
import jax.numpy as jnp
from jax import lax
from jax.experimental import pallas as pl
from jax.experimental.pallas import tpu as pltpu
from jax.experimental.pallas import tpu_sc as plsc

_NC = 2
_NS = 16
_NW = _NC * _NS
_L = 16

_B, _S, _V, _P, _D = 128, 512, 30522, 512, 768
_N = _B * _S
_TPW = _N // _NW
_C = 32
_NCHUNK = _TPW // _C
_NJ = _D // _L
_EPS = 1e-12


def _rsqrt(x):
  xhalf = 0.5 * x
  i = lax.bitcast_convert_type(x, jnp.int32)
  i = jnp.int32(0x5F3759DF) - lax.shift_right_arithmetic(i, 1)
  y = lax.bitcast_convert_type(i, jnp.float32)
  for _ in range(3):
    y = y * (1.5 - xhalf * y * y)
  return y


def _body(ids_hbm, tts_hbm, word_hbm, pos_hbm, seg_hbm, gamma_hbm, beta_hbm,
          out_hbm, idx_word, idx_seg, idx_pos, buf, gamma_v, beta_v,
          sem1, sem2, sem3):
  wid = lax.axis_index("s") * _NC + lax.axis_index("c")
  base = wid * _TPW

  pltpu.sync_copy(ids_hbm.at[pl.ds(base, _TPW)], idx_word)
  pltpu.sync_copy(tts_hbm.at[pl.ds(base, _TPW)], idx_seg)
  pltpu.sync_copy(gamma_hbm, gamma_v)
  pltpu.sync_copy(beta_hbm, beta_v)
  for k in range(_S // _L):
    idx_pos[pl.ds(k * _L, _L)] = lax.iota(jnp.int32, _L) + k * _L

  def chunk(c, _):
    rb = c * _C
    pltpu.async_copy(word_hbm.at[idx_word.at[pl.ds(rb, _C)]], buf, sem1).wait()
    d2 = pltpu.async_copy(seg_hbm.at[idx_seg.at[pl.ds(rb, _C)]], buf, sem2,
                          add=True)
    pb = lax.rem(rb, _S)
    d3 = pltpu.async_copy(pos_hbm.at[idx_pos.at[pl.ds(pb, _C)]], buf, sem3,
                          add=True)
    d2.wait()
    d3.wait()

    def row(r, _):
      sum_v = jnp.zeros((_L,), jnp.float32)
      sq_v = jnp.zeros((_L,), jnp.float32)
      for j in range(_NJ):
        x = buf[r, pl.ds(j * _L, _L)]
        sum_v = sum_v + x
        sq_v = sq_v + x * x
      s1 = jnp.sum(sum_v)
      s2 = jnp.sum(sq_v)
      mean = s1 * (1.0 / _D)
      var = s2 * (1.0 / _D) - mean * mean
      rstd = _rsqrt(var + _EPS)
      mean_b = jnp.broadcast_to(mean, (_L,))
      rstd_b = jnp.broadcast_to(rstd, (_L,))
      for j in range(_NJ):
        sl = pl.ds(j * _L, _L)
        x = buf[r, sl]
        buf[r, sl] = (x - mean_b) * rstd_b * gamma_v[sl] + beta_v[sl]
      return 0

    lax.fori_loop(0, _C, row, 0)
    pltpu.sync_copy(buf, out_hbm.at[pl.ds(base + rb, _C)])
    return 0

  lax.fori_loop(0, _NCHUNK, chunk, 0)


@jax.jit
def _run(ids, tts, word_emb, pos_emb, seg_emb, ln_gamma, ln_beta):
  mesh = plsc.VectorSubcoreMesh(core_axis_name="c", subcore_axis_name="s",
                                num_cores=_NC, num_subcores=_NS)
  f = pl.kernel(
      _body,
      out_type=jax.ShapeDtypeStruct((_N, _D), jnp.float32),
      mesh=mesh,
      compiler_params=pltpu.CompilerParams(needs_layout_passes=False),
      scratch_types=[
          pltpu.VMEM((_TPW,), jnp.int32),
          pltpu.VMEM((_TPW,), jnp.int32),
          pltpu.VMEM((_S,), jnp.int32),
          pltpu.VMEM((_C, _D), jnp.float32),
          pltpu.VMEM((_D,), jnp.float32),
          pltpu.VMEM((_D,), jnp.float32),
          pltpu.SemaphoreType.DMA,
          pltpu.SemaphoreType.DMA,
          pltpu.SemaphoreType.DMA,
      ],
  )
  return f(ids, tts, word_emb, pos_emb, seg_emb, ln_gamma, ln_beta)


def kernel(input_ids, token_type_ids, word_emb, pos_emb, seg_emb, ln_gamma,
           ln_beta):
  ids = input_ids.reshape(_N).astype(jnp.int32)
  tts = token_type_ids.reshape(_N).astype(jnp.int32)
  out = _run(ids, tts, word_emb, pos_emb, seg_emb, ln_gamma, ln_beta)
  return out.reshape(_B, _S, _D)
